# baseline (speedup 1.0000x reference)
import functools
import jax
import jax.numpy as jnp
from jax import lax
from jax.experimental import pallas as pl
from jax.experimental.pallas import tpu as pltpu
from jax.experimental.pallas import tpu_sc as plsc

N_ENTITIES = 1000000
N_DIMS = 64
BATCH = 16384
_TCOL = 128
_NTC = 7813

_info = plsc.get_sparse_core_info()
_NC, _NS, _L = _info.num_cores, _info.num_subcores, _info.num_lanes
_NW = _NC * _NS
_BASE_COLS = _NTC // _NW
_EXTRA = _NTC - _BASE_COLS * _NW
_CH_COLS = 4
_CH_ENT = _CH_COLS * _TCOL
_NCHUNK = (_BASE_COLS + 1 + _CH_COLS - 1) // _CH_COLS
_NSEG = 4
_SEG = 192
_WCAP = _NSEG * _SEG
_SGRP = _SEG // _L
_SLOTS = 32
_RING = 4
_RBYTES = _SLOTS * N_DIMS
_SENT = 0x7FFFFFF0
_FLAT = BATCH * N_DIMS + _NW * _SLOTS * N_DIMS

_mesh = plsc.VectorSubcoreMesh(core_axis_name="c", subcore_axis_name="s")


@functools.partial(
    pl.kernel,
    mesh=_mesh,
    compiler_params=pltpu.CompilerParams(needs_layout_passes=False),
    out_type=jax.ShapeDtypeStruct((_FLAT,), jnp.float32),
    scratch_types=[
        pltpu.VMEM((BATCH,), jnp.int32),
        pltpu.VMEM((_WCAP,), jnp.int32),
        pltpu.VMEM((_WCAP,), jnp.int32),
        pltpu.VMEM((_SLOTS,), jnp.int32),
        pltpu.VMEM((_SLOTS,), jnp.int32),
        pltpu.VMEM((2, N_DIMS, _CH_ENT), jnp.float32),
        pltpu.VMEM((_RING * _RBYTES,), jnp.float32),
        pltpu.SemaphoreType.DMA,
        pltpu.SemaphoreType.DMA,
    ],
)
def _gather_kernel(idx_hbm, tt_hbm, out_hbm, idx_v, wl_e, wl_p, sl_e, sl_p,
                   chunk, ring, sem, osem):
    wid = lax.axis_index("c") * _NS + lax.axis_index("s")
    start = wid * _BASE_COLS + jnp.minimum(wid, _EXTRA)
    ncol = _BASE_COLS + (wid < _EXTRA).astype(jnp.int32)
    wlo = start * _TCOL
    whi = (start + ncol) * _TCOL
    scrap = BATCH + wid * _SLOTS

    pltpu.sync_copy(idx_hbm, idx_v)
    lanes = lax.iota(jnp.int32, _L)

    _QG = BATCH // _L // _NSEG

    def scan_group(g, carry):
        new = []
        for q in range(_NSEG):
            cnt = carry[q]
            gg = g + q * _QG
            v = idx_v[pl.ds(gg * _L, _L)]
            m = (v >= wlo) & (v < whi)
            cum = plsc.cumsum(m.astype(jnp.int32))
            dest = jnp.minimum(cnt + cum - 1, _SEG - 1) + q * _SEG
            plsc.store_scatter(wl_e, [dest], v, mask=m)
            plsc.store_scatter(wl_p, [dest], gg * _L + lanes, mask=m)
            new.append(cnt + cum[_L - 1])
        return tuple(new)

    zero = jnp.int32(0)
    segn = pl.loop(0, _QG, init_carry=(zero,) * _NSEG)(scan_group)

    def fire(h):
        colf = pl.multiple_of(
            (jnp.minimum(start + h * _CH_COLS, start + ncol - _CH_COLS))
            * _TCOL, _TCOL)
        pltpu.async_copy(
            tt_hbm.at[:, pl.ds(colf, _CH_ENT)],
            chunk.at[lax.rem(h, 2)],
            sem,
        )

    fire(0)

    def chunk_body(h):
        par = lax.rem(h, 2)
        rpar = lax.rem(h, _RING)
        a_lo = (start + h * _CH_COLS) * _TCOL
        a_hi = jnp.minimum(a_lo + _CH_ENT, whi)
        colf = jnp.minimum(start + h * _CH_COLS, start + ncol - _CH_COLS) \
            * _TCOL

        @pl.when(h + 1 < _NCHUNK)
        def _():
            fire(h + 1)

        sl_e[pl.ds(0, _L)] = jnp.full((_L,), _SENT, jnp.int32)
        sl_e[pl.ds(_L, _L)] = jnp.full((_L,), _SENT, jnp.int32)

        def mk_bucket(q):
            def bucket_group(g, scnt):
                base = q * _SEG + g * _L
                e = wl_e[pl.ds(base, _L)]
                p = wl_p[pl.ds(base, _L)]
                m = ((g * _L + lanes) < segn[q]) & (e >= a_lo) & (e < a_hi)
                cum = plsc.cumsum(m.astype(jnp.int32))
                dest = jnp.minimum(scnt + cum - 1, _SLOTS - 1)
                plsc.store_scatter(sl_e, [dest], e, mask=m)
                plsc.store_scatter(sl_p, [dest], p, mask=m)
                return scnt + cum[_L - 1]
            return bucket_group

        scnt = zero
        for q in range(_NSEG):
            ub = lax.shift_right_logical(segn[q] + (_L - 1), 4)
            scnt = pl.loop(0, ub, init_carry=scnt)(mk_bucket(q))

        pltpu.make_async_copy(
            tt_hbm.at[:, pl.ds(0, _CH_ENT)], chunk.at[par], sem
        ).wait()

        @pl.when(h >= _RING)
        def _():
            pltpu.make_async_copy(
                out_hbm.at[pl.ds(0, _RBYTES)],
                ring.at[pl.ds(rpar * _RBYTES, _RBYTES)],
                osem,
            ).wait()

        rbase = rpar * _RBYTES
        for half in range(2):
            ve = sl_e[pl.ds(half * _L, _L)]
            vp = sl_p[pl.ds(half * _L, _L)]
            cc = jnp.clip(ve - colf, 0, _CH_ENT - 1)
            sbase = (half * _L + lanes) * N_DIMS
            offv = jnp.where(ve < a_hi, vp, scrap + half * _L + lanes)
            for d in range(N_DIMS):
                vals = plsc.load_gather(
                    chunk.at[par], [jnp.full((_L,), d, jnp.int32), cc])
                plsc.store_scatter(ring, [rbase + sbase + d], vals)
            for s in range(_L):
                slot = half * _L + s
                pltpu.async_copy(
                    ring.at[pl.ds(rbase + slot * N_DIMS, N_DIMS)],
                    out_hbm.at[pl.ds(offv[s] * N_DIMS, N_DIMS)],
                    osem,
                )

    pl.loop(0, _NCHUNK)(chunk_body)
    for _ in range(_RING):
        pltpu.make_async_copy(
            out_hbm.at[pl.ds(0, _RBYTES)],
            ring.at[pl.ds(0, _RBYTES)],
            osem,
        ).wait()


_B_PER_W = BATCH // _NW


@functools.partial(
    pl.kernel,
    mesh=_mesh,
    compiler_params=pltpu.CompilerParams(needs_layout_passes=False),
    out_type=jax.ShapeDtypeStruct((N_DIMS, BATCH), jnp.float32),
    scratch_types=[
        pltpu.VMEM((_B_PER_W * N_DIMS,), jnp.float32),
        pltpu.VMEM((N_DIMS, _B_PER_W), jnp.float32),
    ],
)
def _transpose_kernel(flat_hbm, out_hbm, rows_v, stage):
    wid = lax.axis_index("c") * _NS + lax.axis_index("s")
    base = wid * _B_PER_W
    pltpu.sync_copy(flat_hbm.at[pl.ds(base * N_DIMS, _B_PER_W * N_DIMS)],
                    rows_v)
    lanes = lax.iota(jnp.int32, _L)

    def group(g):
        j16 = g * _L + lanes
        for d in range(N_DIMS):
            vals = plsc.load_gather(rows_v, [j16 * N_DIMS + d])
            plsc.store_scatter(
                stage, [jnp.full((_L,), d, jnp.int32), j16], vals)

    pl.loop(0, _B_PER_W // _L)(group)
    pltpu.sync_copy(stage, out_hbm.at[:, pl.ds(base, _B_PER_W)])


def kernel(inputs, entity_table, relation_table):
    del relation_table
    flat = _gather_kernel(inputs.astype(jnp.int32), entity_table.T)
    return _transpose_kernel(flat).T


# speedup vs baseline: 1.1544x; 1.1544x over previous
import functools
import jax
import jax.numpy as jnp
from jax import lax
from jax.experimental import pallas as pl
from jax.experimental.pallas import tpu as pltpu
from jax.experimental.pallas import tpu_sc as plsc

N_ENTITIES = 1000000
N_DIMS = 64
BATCH = 16384
_TCOL = 128
_NTC = 7813

_info = plsc.get_sparse_core_info()
_NC, _NS, _L = _info.num_cores, _info.num_subcores, _info.num_lanes
_NW = _NC * _NS
_BASE_COLS = _NTC // _NW
_EXTRA = _NTC - _BASE_COLS * _NW
_CH_COLS = 6
_CH_ENT = _CH_COLS * _TCOL
_NCHUNK = (_BASE_COLS + 1 + _CH_COLS - 1) // _CH_COLS
_NSEG = 4
_SEG = 192
_WCAP = _NSEG * _SEG
_SGRP = _SEG // _L
_SLOTS = 48
_RING = 4
_RBYTES = _SLOTS * N_DIMS
_SENT = 0x7FFFFFF0
_FLAT = BATCH * N_DIMS + _NW * _SLOTS * N_DIMS

_mesh = plsc.VectorSubcoreMesh(core_axis_name="c", subcore_axis_name="s")


@functools.partial(
    pl.kernel,
    mesh=_mesh,
    compiler_params=pltpu.CompilerParams(needs_layout_passes=False),
    out_type=jax.ShapeDtypeStruct((_FLAT,), jnp.float32),
    scratch_types=[
        pltpu.VMEM((BATCH,), jnp.int32),
        pltpu.VMEM((_WCAP,), jnp.int32),
        pltpu.VMEM((_WCAP,), jnp.int32),
        pltpu.VMEM((_SLOTS,), jnp.int32),
        pltpu.VMEM((_SLOTS,), jnp.int32),
        pltpu.VMEM((2, N_DIMS, _CH_ENT), jnp.float32),
        pltpu.VMEM((_RING * _RBYTES,), jnp.float32),
        pltpu.SemaphoreType.DMA,
        pltpu.SemaphoreType.DMA,
    ],
)
def _gather_kernel(idx_hbm, tt_hbm, out_hbm, idx_v, wl_e, wl_p, sl_e, sl_p,
                   chunk, ring, sem, osem):
    wid = lax.axis_index("c") * _NS + lax.axis_index("s")
    start = wid * _BASE_COLS + jnp.minimum(wid, _EXTRA)
    ncol = _BASE_COLS + (wid < _EXTRA).astype(jnp.int32)
    wlo = start * _TCOL
    whi = (start + ncol) * _TCOL
    scrap = BATCH + wid * _SLOTS

    pltpu.sync_copy(idx_hbm, idx_v)
    lanes = lax.iota(jnp.int32, _L)

    _QG = BATCH // _L // _NSEG

    def scan_group(g, carry):
        new = []
        for q in range(_NSEG):
            cnt = carry[q]
            gg = g + q * _QG
            v = idx_v[pl.ds(gg * _L, _L)]
            m = (v >= wlo) & (v < whi)
            cum = plsc.cumsum(m.astype(jnp.int32))
            dest = jnp.minimum(cnt + cum - 1, _SEG - 1) + q * _SEG
            plsc.store_scatter(wl_e, [dest], v, mask=m)
            plsc.store_scatter(wl_p, [dest], gg * _L + lanes, mask=m)
            new.append(cnt + cum[_L - 1])
        return tuple(new)

    zero = jnp.int32(0)
    segn = pl.loop(0, _QG, init_carry=(zero,) * _NSEG)(scan_group)

    def fire(h):
        colf = pl.multiple_of(
            (jnp.minimum(start + h * _CH_COLS, start + ncol - _CH_COLS))
            * _TCOL, _TCOL)
        pltpu.async_copy(
            tt_hbm.at[:, pl.ds(colf, _CH_ENT)],
            chunk.at[lax.rem(h, 2)],
            sem,
        )

    fire(0)

    def chunk_body(h):
        par = lax.rem(h, 2)
        rpar = lax.rem(h, _RING)
        a_lo = (start + h * _CH_COLS) * _TCOL
        a_hi = jnp.minimum(a_lo + _CH_ENT, whi)
        colf = jnp.minimum(start + h * _CH_COLS, start + ncol - _CH_COLS) \
            * _TCOL

        @pl.when(h + 1 < _NCHUNK)
        def _():
            fire(h + 1)

        for half in range(_SLOTS // _L):
            sl_e[pl.ds(half * _L, _L)] = jnp.full((_L,), _SENT, jnp.int32)

        def mk_bucket(q):
            def bucket_group(g, scnt):
                base = q * _SEG + g * _L
                e = wl_e[pl.ds(base, _L)]
                p = wl_p[pl.ds(base, _L)]
                m = ((g * _L + lanes) < segn[q]) & (e >= a_lo) & (e < a_hi)
                cum = plsc.cumsum(m.astype(jnp.int32))
                dest = jnp.minimum(scnt + cum - 1, _SLOTS - 1)
                plsc.store_scatter(sl_e, [dest], e, mask=m)
                plsc.store_scatter(sl_p, [dest], p, mask=m)
                return scnt + cum[_L - 1]
            return bucket_group

        scnt = zero
        for q in range(_NSEG):
            ub = lax.shift_right_logical(segn[q] + (_L - 1), 4)
            scnt = pl.loop(0, ub, init_carry=scnt)(mk_bucket(q))

        pltpu.make_async_copy(
            tt_hbm.at[:, pl.ds(0, _CH_ENT)], chunk.at[par], sem
        ).wait()

        @pl.when(h >= _RING)
        def _():
            pltpu.make_async_copy(
                out_hbm.at[pl.ds(0, _RBYTES)],
                ring.at[pl.ds(rpar * _RBYTES, _RBYTES)],
                osem,
            ).wait()

        rbase = rpar * _RBYTES
        for half in range(_SLOTS // _L):
            ve = sl_e[pl.ds(half * _L, _L)]
            vp = sl_p[pl.ds(half * _L, _L)]
            cc = jnp.clip(ve - colf, 0, _CH_ENT - 1)
            sbase = (half * _L + lanes) * N_DIMS
            offv = jnp.where(ve < a_hi, vp, scrap + half * _L + lanes)
            for d in range(N_DIMS):
                vals = plsc.load_gather(
                    chunk.at[par], [jnp.full((_L,), d, jnp.int32), cc])
                plsc.store_scatter(ring, [rbase + sbase + d], vals)
            for s in range(_L):
                slot = half * _L + s
                pltpu.async_copy(
                    ring.at[pl.ds(rbase + slot * N_DIMS, N_DIMS)],
                    out_hbm.at[pl.ds(offv[s] * N_DIMS, N_DIMS)],
                    osem,
                )

    pl.loop(0, _NCHUNK)(chunk_body)
    for _ in range(_RING):
        pltpu.make_async_copy(
            out_hbm.at[pl.ds(0, _RBYTES)],
            ring.at[pl.ds(0, _RBYTES)],
            osem,
        ).wait()


def kernel(inputs, entity_table, relation_table):
    del relation_table
    flat = _gather_kernel(inputs.astype(jnp.int32), entity_table.T)
    return flat[: BATCH * N_DIMS].reshape(BATCH, N_DIMS)


# speedup vs baseline: 1.3429x; 1.1634x over previous
import functools
import jax
import jax.numpy as jnp
from jax import lax
from jax.experimental import pallas as pl
from jax.experimental.pallas import tpu as pltpu
from jax.experimental.pallas import tpu_sc as plsc

N_ENTITIES = 1000000
N_DIMS = 64
BATCH = 16384
_TCOL = 128
_NTC = 7813

_info = plsc.get_sparse_core_info()
_NC, _NS, _L = _info.num_cores, _info.num_subcores, _info.num_lanes
_NW = _NC * _NS
_BASE_COLS = _NTC // _NW
_EXTRA = _NTC - _BASE_COLS * _NW
_CH_COLS = 6
_CH_ENT = _CH_COLS * _TCOL
_NCHUNK = (_BASE_COLS + 1 + _CH_COLS - 1) // _CH_COLS
_NSEG = 4
_SEG = 192
_WCAP = _NSEG * _SEG
_SGRP = _SEG // _L
_SLOTS = 48
_RING = 4
_RBYTES = _SLOTS * N_DIMS
_SENT = 0x7FFFFFF0
_FLAT = BATCH * N_DIMS + _NW * _SLOTS * N_DIMS

_mesh = plsc.VectorSubcoreMesh(core_axis_name="c", subcore_axis_name="s")


@functools.partial(
    pl.kernel,
    mesh=_mesh,
    compiler_params=pltpu.CompilerParams(needs_layout_passes=False),
    out_type=jax.ShapeDtypeStruct((_FLAT,), jnp.float32),
    scratch_types=[
        pltpu.VMEM((BATCH,), jnp.int32),
        pltpu.VMEM((_WCAP,), jnp.int32),
        pltpu.VMEM((_WCAP,), jnp.int32),
        pltpu.VMEM((_SLOTS,), jnp.int32),
        pltpu.VMEM((_SLOTS,), jnp.int32),
        pltpu.VMEM((2, N_DIMS, _CH_ENT), jnp.float32),
        pltpu.VMEM((_RING * _RBYTES,), jnp.float32),
        pltpu.SemaphoreType.DMA,
        pltpu.SemaphoreType.DMA,
    ],
)
def _gather_kernel(idx_hbm, tt_hbm, out_hbm, idx_v, wl_e, wl_p, sl_e, sl_p,
                   chunk, ring, sem, osem):
    wid = lax.axis_index("c") * _NS + lax.axis_index("s")
    start = wid * _BASE_COLS + jnp.minimum(wid, _EXTRA)
    ncol = _BASE_COLS + (wid < _EXTRA).astype(jnp.int32)
    wlo = start * _TCOL
    whi = (start + ncol) * _TCOL
    scrap = BATCH + wid * _SLOTS

    pltpu.sync_copy(idx_hbm, idx_v)
    lanes = lax.iota(jnp.int32, _L)

    _QG = BATCH // _L // _NSEG

    def scan_group(g, carry):
        new = []
        for q in range(_NSEG):
            cnt = carry[q]
            gg = g + q * _QG
            v = idx_v[pl.ds(gg * _L, _L)]
            m = (v >= wlo) & (v < whi)
            cum = plsc.cumsum(m.astype(jnp.int32))
            dest = jnp.minimum(cnt + cum - 1, _SEG - 1) + q * _SEG
            plsc.store_scatter(wl_e, [dest], v, mask=m)
            plsc.store_scatter(wl_p, [dest], gg * _L + lanes, mask=m)
            new.append(cnt + cum[_L - 1])
        return tuple(new)

    zero = jnp.int32(0)
    segn = pl.loop(0, _QG, init_carry=(zero,) * _NSEG)(scan_group)

    def fire(h):
        colf = pl.multiple_of(
            (jnp.minimum(start + h * _CH_COLS, start + ncol - _CH_COLS))
            * _TCOL, _TCOL)
        pltpu.async_copy(
            tt_hbm.at[:, pl.ds(colf, _CH_ENT)],
            chunk.at[lax.rem(h, 2)],
            sem,
        )

    fire(0)

    def chunk_body(h):
        par = lax.rem(h, 2)
        rpar = lax.rem(h, _RING)
        a_lo = (start + h * _CH_COLS) * _TCOL
        a_hi = jnp.minimum(a_lo + _CH_ENT, whi)
        colf = jnp.minimum(start + h * _CH_COLS, start + ncol - _CH_COLS) \
            * _TCOL

        @pl.when(h + 1 < _NCHUNK)
        def _():
            fire(h + 1)

        for half in range(_SLOTS // _L):
            sl_e[pl.ds(half * _L, _L)] = jnp.full((_L,), _SENT, jnp.int32)

        def mk_bucket(q):
            def bucket_group(g, scnt):
                base = q * _SEG + g * _L
                e = wl_e[pl.ds(base, _L)]
                p = wl_p[pl.ds(base, _L)]
                m = ((g * _L + lanes) < segn[q]) & (e >= a_lo) & (e < a_hi)
                cum = plsc.cumsum(m.astype(jnp.int32))
                dest = jnp.minimum(scnt + cum - 1, _SLOTS - 1)
                plsc.store_scatter(sl_e, [dest], e, mask=m)
                plsc.store_scatter(sl_p, [dest], p, mask=m)
                return scnt + cum[_L - 1]
            return bucket_group

        scnt = zero
        for q in range(_NSEG):
            ub = lax.shift_right_logical(segn[q] + (_L - 1), 4)
            scnt = pl.loop(0, ub, init_carry=scnt)(mk_bucket(q))

        pltpu.make_async_copy(
            tt_hbm.at[:, pl.ds(0, _CH_ENT)], chunk.at[par], sem
        ).wait()

        @pl.when(h >= _RING)
        def _():
            pltpu.make_async_copy(
                out_hbm.at[pl.ds(0, _RBYTES)],
                ring.at[pl.ds(rpar * _RBYTES, _RBYTES)],
                osem,
            ).wait()

        rbase = rpar * _RBYTES
        for half in range(_SLOTS // _L):
            ve = sl_e[pl.ds(half * _L, _L)]
            vp = sl_p[pl.ds(half * _L, _L)]
            cc = jnp.clip(ve - colf, 0, _CH_ENT - 1)
            sbase = (half * _L + lanes) * N_DIMS
            offv = jnp.where(ve < a_hi, vp, scrap + half * _L + lanes)

            def gather_half(cc=cc, sbase=sbase):
                for d in range(N_DIMS):
                    vals = plsc.load_gather(
                        chunk.at[par], [jnp.full((_L,), d, jnp.int32), cc])
                    plsc.store_scatter(ring, [rbase + sbase + d], vals)

            if half == 0:
                gather_half()
            else:
                pl.when(scnt > half * _L)(gather_half)
            for s in range(_L):
                slot = half * _L + s
                pltpu.async_copy(
                    ring.at[pl.ds(rbase + slot * N_DIMS, N_DIMS)],
                    out_hbm.at[pl.ds(offv[s] * N_DIMS, N_DIMS)],
                    osem,
                )

    pl.loop(0, _NCHUNK)(chunk_body)
    for _ in range(_RING):
        pltpu.make_async_copy(
            out_hbm.at[pl.ds(0, _RBYTES)],
            ring.at[pl.ds(0, _RBYTES)],
            osem,
        ).wait()


def kernel(inputs, entity_table, relation_table):
    del relation_table
    flat = _gather_kernel(inputs.astype(jnp.int32), entity_table.T)
    return flat[: BATCH * N_DIMS].reshape(BATCH, N_DIMS)


# speedup vs baseline: 1.3566x; 1.0102x over previous
import functools
import jax
import jax.numpy as jnp
from jax import lax
from jax.experimental import pallas as pl
from jax.experimental.pallas import tpu as pltpu
from jax.experimental.pallas import tpu_sc as plsc

N_ENTITIES = 1000000
N_DIMS = 64
BATCH = 16384
_TCOL = 128
_NTC = 7813

_info = plsc.get_sparse_core_info()
_NC, _NS, _L = _info.num_cores, _info.num_subcores, _info.num_lanes
_NW = _NC * _NS
_BASE_COLS = _NTC // _NW
_EXTRA = _NTC - _BASE_COLS * _NW
_CH_COLS = 6
_CH_ENT = _CH_COLS * _TCOL
_NCHUNK = (_BASE_COLS + 1 + _CH_COLS - 1) // _CH_COLS
_NSEG = 4
_SEG = 192
_WCAP = _NSEG * _SEG
_SGRP = _SEG // _L
_SLOTS = 48
_RING = 4
_RBYTES = _SLOTS * N_DIMS
_SENT = 0x7FFFFFF0
_FLAT = BATCH * N_DIMS + _NW * _SLOTS * N_DIMS

_mesh = plsc.VectorSubcoreMesh(core_axis_name="c", subcore_axis_name="s")


@functools.partial(
    pl.kernel,
    mesh=_mesh,
    compiler_params=pltpu.CompilerParams(needs_layout_passes=False),
    out_type=jax.ShapeDtypeStruct((_FLAT,), jnp.float32),
    scratch_types=[
        pltpu.VMEM((BATCH,), jnp.int32),
        pltpu.VMEM((_WCAP,), jnp.int32),
        pltpu.VMEM((_WCAP,), jnp.int32),
        pltpu.VMEM((_SLOTS,), jnp.int32),
        pltpu.VMEM((_SLOTS,), jnp.int32),
        pltpu.VMEM((2, N_DIMS, _CH_ENT), jnp.float32),
        pltpu.VMEM((_RING * _RBYTES,), jnp.float32),
        pltpu.SemaphoreType.DMA,
        pltpu.SemaphoreType.DMA,
    ],
)
def _gather_kernel(idx_hbm, tt_hbm, out_hbm, idx_v, wl_e, wl_p, sl_e, sl_p,
                   chunk, ring, sem, osem):
    wid = lax.axis_index("c") * _NS + lax.axis_index("s")
    start = wid * _BASE_COLS + jnp.minimum(wid, _EXTRA)
    ncol = _BASE_COLS + (wid < _EXTRA).astype(jnp.int32)
    wlo = start * _TCOL
    whi = (start + ncol) * _TCOL
    scrap = BATCH + wid * _SLOTS

    def fire(h):
        colf = pl.multiple_of(
            (jnp.minimum(start + h * _CH_COLS, start + ncol - _CH_COLS))
            * _TCOL, _TCOL)
        pltpu.async_copy(
            tt_hbm.at[:, pl.ds(colf, _CH_ENT)],
            chunk.at[lax.rem(h, 2)],
            sem,
        )

    fire(0)
    fire(1)
    pltpu.sync_copy(idx_hbm, idx_v)
    lanes = lax.iota(jnp.int32, _L)

    _QG = BATCH // _L // _NSEG

    def scan_group(g, carry):
        new = []
        for q in range(_NSEG):
            cnt = carry[q]
            gg = g + q * _QG
            v = idx_v[pl.ds(gg * _L, _L)]
            m = (v >= wlo) & (v < whi)
            cum = plsc.cumsum(m.astype(jnp.int32))
            dest = jnp.minimum(cnt + cum - 1, _SEG - 1) + q * _SEG
            plsc.store_scatter(wl_e, [dest], v, mask=m)
            plsc.store_scatter(wl_p, [dest], gg * _L + lanes, mask=m)
            new.append(cnt + cum[_L - 1])
        return tuple(new)

    zero = jnp.int32(0)
    segn = pl.loop(0, _QG, init_carry=(zero,) * _NSEG)(scan_group)

    def chunk_body(h):
        par = lax.rem(h, 2)
        rpar = lax.rem(h, _RING)
        a_lo = (start + h * _CH_COLS) * _TCOL
        a_hi = jnp.minimum(a_lo + _CH_ENT, whi)
        colf = jnp.minimum(start + h * _CH_COLS, start + ncol - _CH_COLS) \
            * _TCOL

        for half in range(_SLOTS // _L):
            sl_e[pl.ds(half * _L, _L)] = jnp.full((_L,), _SENT, jnp.int32)

        def mk_bucket(q):
            def bucket_group(g, scnt):
                base = q * _SEG + g * _L
                e = wl_e[pl.ds(base, _L)]
                p = wl_p[pl.ds(base, _L)]
                m = ((g * _L + lanes) < segn[q]) & (e >= a_lo) & (e < a_hi)
                cum = plsc.cumsum(m.astype(jnp.int32))
                dest = jnp.minimum(scnt + cum - 1, _SLOTS - 1)
                plsc.store_scatter(sl_e, [dest], e, mask=m)
                plsc.store_scatter(sl_p, [dest], p, mask=m)
                return scnt + cum[_L - 1]
            return bucket_group

        scnt = zero
        for q in range(_NSEG):
            ub = lax.shift_right_logical(segn[q] + (_L - 1), 4)
            scnt = pl.loop(0, ub, init_carry=scnt)(mk_bucket(q))

        pltpu.make_async_copy(
            tt_hbm.at[:, pl.ds(0, _CH_ENT)], chunk.at[par], sem
        ).wait()

        @pl.when(h >= _RING)
        def _():
            pltpu.make_async_copy(
                out_hbm.at[pl.ds(0, _RBYTES)],
                ring.at[pl.ds(rpar * _RBYTES, _RBYTES)],
                osem,
            ).wait()

        rbase = rpar * _RBYTES
        for half in range(_SLOTS // _L):
            ve = sl_e[pl.ds(half * _L, _L)]
            vp = sl_p[pl.ds(half * _L, _L)]
            cc = jnp.clip(ve - colf, 0, _CH_ENT - 1)
            sbase = (half * _L + lanes) * N_DIMS
            offv = jnp.where(ve < a_hi, vp, scrap + half * _L + lanes)

            def live_half(cc=cc, sbase=sbase, offv=offv, half=half):
                for d in range(N_DIMS):
                    vals = plsc.load_gather(
                        chunk.at[par], [jnp.full((_L,), d, jnp.int32), cc])
                    plsc.store_scatter(ring, [rbase + sbase + d], vals)
                for s in range(_L):
                    slot = half * _L + s
                    pltpu.async_copy(
                        ring.at[pl.ds(rbase + slot * N_DIMS, N_DIMS)],
                        out_hbm.at[pl.ds(offv[s] * N_DIMS, N_DIMS)],
                        osem,
                    )

            def scrap_half(half=half):
                pltpu.async_copy(
                    ring.at[pl.ds(rbase + half * _L * N_DIMS, _L * N_DIMS)],
                    out_hbm.at[pl.ds((scrap + half * _L) * N_DIMS,
                                     _L * N_DIMS)],
                    osem,
                )

            if half == 0:
                live_half()
            else:
                pl.when(scnt > half * _L)(live_half)
                pl.when(scnt <= half * _L)(scrap_half)

        @pl.when(h + 2 < _NCHUNK)
        def _():
            fire(h + 2)

    pl.loop(0, _NCHUNK)(chunk_body)
    for _ in range(_RING):
        pltpu.make_async_copy(
            out_hbm.at[pl.ds(0, _RBYTES)],
            ring.at[pl.ds(0, _RBYTES)],
            osem,
        ).wait()


def kernel(inputs, entity_table, relation_table):
    del relation_table
    flat = _gather_kernel(inputs.astype(jnp.int32), entity_table.T)
    return flat[: BATCH * N_DIMS].reshape(BATCH, N_DIMS)


# speedup vs baseline: 1.4104x; 1.0396x over previous
import functools
import jax
import jax.numpy as jnp
from jax import lax
from jax.experimental import pallas as pl
from jax.experimental.pallas import tpu as pltpu
from jax.experimental.pallas import tpu_sc as plsc

N_ENTITIES = 1000000
N_DIMS = 64
BATCH = 16384
_TCOL = 128
_NTC = 7813

_info = plsc.get_sparse_core_info()
_NC, _NS, _L = _info.num_cores, _info.num_subcores, _info.num_lanes
_NW = _NC * _NS
_BASE_COLS = _NTC // _NW
_EXTRA = _NTC - _BASE_COLS * _NW
_CH_COLS = 6
_CH_ENT = _CH_COLS * _TCOL
_NCHUNK = (_BASE_COLS + 1 + _CH_COLS - 1) // _CH_COLS
_NSEG = 4
_SEG = 192
_WCAP = _NSEG * _SEG
_SGRP = _SEG // _L
_SLOTS = 48
_RING = 4
_RBYTES = _SLOTS * N_DIMS
_SENT = 0x7FFFFFF0
_FLAT = BATCH * N_DIMS + _NW * _SLOTS * N_DIMS

_mesh = plsc.VectorSubcoreMesh(core_axis_name="c", subcore_axis_name="s")


@functools.partial(
    pl.kernel,
    mesh=_mesh,
    compiler_params=pltpu.CompilerParams(needs_layout_passes=False),
    out_type=jax.ShapeDtypeStruct((_FLAT,), jnp.float32),
    scratch_types=[
        pltpu.VMEM((BATCH,), jnp.int32),
        pltpu.VMEM((_WCAP,), jnp.int32),
        pltpu.VMEM((_WCAP,), jnp.int32),
        pltpu.VMEM((_SLOTS,), jnp.int32),
        pltpu.VMEM((_SLOTS,), jnp.int32),
        pltpu.VMEM((2, N_DIMS, _CH_ENT), jnp.float32),
        pltpu.VMEM((_RING * _RBYTES,), jnp.float32),
        pltpu.SemaphoreType.DMA,
        pltpu.SemaphoreType.DMA,
    ],
)
def _gather_kernel(idx_hbm, tt_hbm, out_hbm, idx_v, wl_e, wl_p, sl_e, sl_p,
                   chunk, ring, sem, osem):
    wid = lax.axis_index("c") * _NS + lax.axis_index("s")
    start = wid * _BASE_COLS + jnp.minimum(wid, _EXTRA)
    ncol = _BASE_COLS + (wid < _EXTRA).astype(jnp.int32)
    wlo = start * _TCOL
    whi = (start + ncol) * _TCOL
    scrap = BATCH + wid * _SLOTS

    def fire(h):
        colf = pl.multiple_of(
            (jnp.minimum(start + h * _CH_COLS, start + ncol - _CH_COLS))
            * _TCOL, _TCOL)
        pltpu.async_copy(
            tt_hbm.at[:, pl.ds(colf, _CH_ENT)],
            chunk.at[lax.rem(h, 2)],
            sem,
        )

    fire(0)
    fire(1)
    pltpu.sync_copy(idx_hbm, idx_v)
    lanes = lax.iota(jnp.int32, _L)

    _QG = BATCH // _L // _NSEG

    def scan_group(g, carry):
        new = []
        for q in range(_NSEG):
            cnt = carry[q]
            gg = g + q * _QG
            v = idx_v[pl.ds(gg * _L, _L)]
            m = (v >= wlo) & (v < whi)
            cum = plsc.cumsum(m.astype(jnp.int32))
            dest = jnp.minimum(cnt + cum - 1, _SEG - 1) + q * _SEG
            plsc.store_scatter(wl_e, [dest], v, mask=m)
            plsc.store_scatter(wl_p, [dest], gg * _L + lanes, mask=m)
            new.append(cnt + cum[_L - 1])
        return tuple(new)

    zero = jnp.int32(0)
    segn = pl.loop(0, _QG, init_carry=(zero,) * _NSEG)(scan_group)

    def chunk_body(h):
        par = lax.rem(h, 2)
        rpar = lax.rem(h, _RING)
        a_lo = (start + h * _CH_COLS) * _TCOL
        a_hi = jnp.minimum(a_lo + _CH_ENT, whi)
        colf = jnp.minimum(start + h * _CH_COLS, start + ncol - _CH_COLS) \
            * _TCOL

        for half in range(_SLOTS // _L):
            sl_e[pl.ds(half * _L, _L)] = jnp.full((_L,), _SENT, jnp.int32)

        def mk_bucket(q):
            def bucket_group(g, scnt):
                base = q * _SEG + g * _L
                e = wl_e[pl.ds(base, _L)]
                p = wl_p[pl.ds(base, _L)]
                m = ((g * _L + lanes) < segn[q]) & (e >= a_lo) & (e < a_hi)
                cum = plsc.cumsum(m.astype(jnp.int32))
                dest = jnp.minimum(scnt + cum - 1, _SLOTS - 1)
                plsc.store_scatter(sl_e, [dest], e, mask=m)
                plsc.store_scatter(sl_p, [dest], p, mask=m)
                return scnt + cum[_L - 1]
            return bucket_group

        scnt = zero
        for q in range(_NSEG):
            ub = lax.shift_right_logical(segn[q] + (_L - 1), 4)
            scnt = pl.loop(0, ub, init_carry=scnt)(mk_bucket(q))

        pltpu.make_async_copy(
            tt_hbm.at[:, pl.ds(0, _CH_ENT)], chunk.at[par], sem
        ).wait()

        @pl.when(h >= _RING)
        def _():
            pltpu.make_async_copy(
                out_hbm.at[pl.ds(0, _RBYTES)],
                ring.at[pl.ds(rpar * _RBYTES, _RBYTES)],
                osem,
            ).wait()

        rbase = rpar * _RBYTES
        for half in range(_SLOTS // _L):
            ve = sl_e[pl.ds(half * _L, _L)]
            vp = sl_p[pl.ds(half * _L, _L)]
            cc = jnp.clip(ve - colf, 0, _CH_ENT - 1)
            sbase = (half * _L + lanes) * N_DIMS
            vps = jnp.where(vp >= BATCH // 2, 2 * vp - (BATCH - 1), 2 * vp)
            offv = jnp.where(ve < a_hi, vps, scrap + half * _L + lanes)

            def live_half(cc=cc, sbase=sbase, offv=offv, half=half):
                for d in range(N_DIMS):
                    vals = plsc.load_gather(
                        chunk.at[par], [jnp.full((_L,), d, jnp.int32), cc])
                    plsc.store_scatter(ring, [rbase + sbase + d], vals)
                for s in range(_L):
                    slot = half * _L + s
                    pltpu.async_copy(
                        ring.at[pl.ds(rbase + slot * N_DIMS, N_DIMS)],
                        out_hbm.at[pl.ds(offv[s] * N_DIMS, N_DIMS)],
                        osem,
                    )

            def scrap_half(half=half):
                pltpu.async_copy(
                    ring.at[pl.ds(rbase + half * _L * N_DIMS, _L * N_DIMS)],
                    out_hbm.at[pl.ds((scrap + half * _L) * N_DIMS,
                                     _L * N_DIMS)],
                    osem,
                )

            if half == 0:
                live_half()
            else:
                pl.when(scnt > half * _L)(live_half)
                pl.when(scnt <= half * _L)(scrap_half)

        @pl.when(h + 2 < _NCHUNK)
        def _():
            fire(h + 2)

    pl.loop(0, _NCHUNK)(chunk_body)
    for _ in range(_RING):
        pltpu.make_async_copy(
            out_hbm.at[pl.ds(0, _RBYTES)],
            ring.at[pl.ds(0, _RBYTES)],
            osem,
        ).wait()


def _tc_transpose_body(x_ref, o_ref):
    half_rows = BATCH // 2
    o_ref[:, 0:half_rows] = x_ref[0:half_rows, 0:N_DIMS].T
    o_ref[:, half_rows:BATCH] = x_ref[0:half_rows, N_DIMS:2 * N_DIMS].T


_tc_transpose = pl.pallas_call(
    _tc_transpose_body,
    out_shape=jax.ShapeDtypeStruct((N_DIMS, BATCH), jnp.float32),
)


def kernel(inputs, entity_table, relation_table):
    del relation_table
    flat = _gather_kernel(inputs.astype(jnp.int32), entity_table.T)
    return _tc_transpose(flat.reshape(_FLAT // 128, 128)).T
